# baseline (speedup 1.0000x reference)
import jax
import jax.numpy as jnp
from jax import lax
from jax.experimental import pallas as pl
from jax.experimental.pallas import tpu as pltpu

_INPUT_SIZE = 64
_N_FRAMES = 2048
_N_LMK = 543
_LIPS = [61, 185, 40, 39, 37, 0, 267, 269, 270, 409, 291, 146, 91, 181, 84,
         17, 314, 405, 321, 375, 78, 191, 80, 81, 82, 13, 312, 311, 310, 415,
         95, 88, 178, 87, 14, 317, 402, 318, 324, 308]
_HANDS = list(range(468, 489)) + list(range(522, 543))
_LANDMARKS = _LIPS + _HANDS
_N_COLS = 82
_TAIL0 = 536
_N_TAIL = _N_LMK - _TAIL0

_D_RUNS = []
_GROW_D = {}
_GD = 0
_DBANDS = [(468, 489), (522, _TAIL0)]
_DWINS = sorted({_l // 8 for _l in _LANDMARKS if _l < _TAIL0})
_GAP = 2
_i = 0
while _i < len(_DWINS):
    _j = _i
    while _j + 1 < len(_DWINS) and _DWINS[_j + 1] - _DWINS[_j] <= _GAP + 1:
        _j += 1
    _start, _nr = _DWINS[_i] * 8, (_DWINS[_j] - _DWINS[_i] + 1) * 8
    _is_hand = any(_start < hi and lo < _start + _nr for lo, hi in _DBANDS)
    _D_RUNS.append((_start, _nr, _GD, _is_hand))
    for _r in range(_start, _start + _nr):
        _GROW_D[_r] = _GD + _r - _start
    _GD += _nr
    _i = _j + 1
_G_TAIL = _GD
for _k in range(_N_TAIL):
    _GROW_D[_TAIL0 + _k] = _G_TAIL + _k
_GD += 8
_GTOT = 3 * _GD

_HAND_SEGS = []
for _d in range(3):
    for _lo, _hi in ((468, 489), (522, 543)):
        _r = _lo
        while _r < _hi:
            _g0, _ln = _GROW_D[_r], 1
            while _r + _ln < _hi and _GROW_D[_r + _ln] == _g0 + _ln:
                _ln += 1
            _HAND_SEGS.append((_d * _GD + _g0, _ln))
            _r += _ln
_SEL_SEGS = []
for _d in range(3):
    _gs = [_d * _GD + _GROW_D[_l] for _l in _LANDMARKS]
    _segs, _k = [], 0
    while _k < len(_gs):
        _g0, _ln = _gs[_k], 1
        while _k + _ln < len(_gs) and _gs[_k + _ln] == _g0 + _ln:
            _ln += 1
        _segs.append((_g0, _ln))
        _k += _ln
    _SEL_SEGS.append(_segs)


def _preprocess_kernel(x_ref, out_ref, nef_ref, g_ref, sem_h, sem_l):
    hand_copies = []
    for start, nr, g0, is_hand in _D_RUNS:
        if not is_hand:
            continue
        for d in range(3):
            c = pltpu.make_async_copy(
                x_ref.at[d, start:start + nr, :],
                g_ref.at[d * _GD + g0:d * _GD + g0 + nr, :], sem_h)
            c.start()
            hand_copies.append(c)
    for d in range(3):
        g0 = d * _GD + _G_TAIL
        g_ref[g0 + _N_TAIL:g0 + 8, :] = jnp.zeros((1, _N_FRAMES), jnp.float32)
        c = pltpu.make_async_copy(x_ref.at[d, _TAIL0:_N_LMK, :],
                                  g_ref.at[g0:g0 + _N_TAIL, :], sem_h)
        c.start()
        hand_copies.append(c)
    lip_copies = []
    for start, nr, g0, is_hand in _D_RUNS:
        if is_hand:
            continue
        for d in range(3):
            c = pltpu.make_async_copy(
                x_ref.at[d, start:start + nr, :],
                g_ref.at[d * _GD + g0:d * _GD + g0 + nr, :], sem_l)
            c.start()
            lip_copies.append(c)
    for c in hand_copies:
        c.wait()

    hs = jnp.zeros((1, _N_FRAMES), jnp.float32)
    for g0, ln in _HAND_SEGS:
        hs = hs + jnp.sum(g_ref[g0:g0 + ln, :], axis=0, keepdims=True)
    m = (hs > 0).astype(jnp.float32)

    p_incl = m
    k = 1
    while k < _N_FRAMES:
        shifted = jnp.concatenate(
            [jnp.zeros((1, k), jnp.float32), p_incl[:, :_N_FRAMES - k]], axis=1)
        p_incl = p_incl + shifted
        k *= 2
    p = p_incl - m
    n = lax.slice(p_incl, (0, _N_FRAMES - 1), (1, _N_FRAMES))

    pool = jnp.floor((2.0 * n + (_INPUT_SIZE - 1)) / _INPUT_SIZE)
    q = pool + 1.0
    pad_left = jnp.floor((pool * _INPUT_SIZE - 2.0 * n) / 2.0) + _INPUT_SIZE // 2
    total = _INPUT_SIZE * q

    i_col = lax.broadcasted_iota(
        jnp.int32, (_INPUT_SIZE, 1), 0).astype(jnp.float32)
    a = jnp.where(p == 0, 0.0, 2.0 * p + pad_left)
    b = jnp.where(p == n - 1.0, total, 2.0 * p + 2.0 + pad_left)
    lo = jnp.maximum(a, i_col * q)
    hi = jnp.minimum(b, (i_col + 1.0) * q)
    w_long = jnp.maximum(hi - lo, 0.0) / q
    w_short = (p == i_col).astype(jnp.float32)
    wt = m * jnp.where(n < _INPUT_SIZE, w_short, w_long)

    f_row = lax.broadcasted_iota(
        jnp.int32, (1, _N_FRAMES), 1).astype(jnp.float32)
    nef = jnp.sum(wt * f_row, axis=1, keepdims=True)
    nef_ref[...] = jnp.where((i_col < n) | (n >= _INPUT_SIZE), nef, -1.0)

    for c in lip_copies:
        c.wait()
    wfi = wt.T
    pooled = lax.dot_general(g_ref[...], wfi, (((1,), (0,)), ((), ())))
    pooled_t = pooled.T
    out_ref[...] = jnp.concatenate(
        [jnp.concatenate([pooled_t[:, g0:g0 + ln] for g0, ln in segs], axis=1)
         for segs in _SEL_SEGS], axis=0)


def kernel(data0):
    X = data0.transpose(2, 1, 0)
    out, nef = pl.pallas_call(
        _preprocess_kernel,
        in_specs=[pl.BlockSpec(memory_space=pl.ANY)],
        out_shape=(
            jax.ShapeDtypeStruct((3 * _INPUT_SIZE, _N_COLS), jnp.float32),
            jax.ShapeDtypeStruct((_INPUT_SIZE, 1), jnp.float32),
        ),
        scratch_shapes=[
            pltpu.VMEM((_GTOT, _N_FRAMES), jnp.float32),
            pltpu.SemaphoreType.DMA,
            pltpu.SemaphoreType.DMA,
        ],
    )(X)
    return (out.reshape(3, _INPUT_SIZE, _N_COLS).transpose(1, 2, 0),
            nef.reshape(_INPUT_SIZE))


# speedup vs baseline: 1.0660x; 1.0660x over previous
import jax
import jax.numpy as jnp
from jax import lax
from jax.experimental import pallas as pl
from jax.experimental.pallas import tpu as pltpu

_INPUT_SIZE = 64
_N_FRAMES = 2048
_N_LMK = 543
_LIPS = [61, 185, 40, 39, 37, 0, 267, 269, 270, 409, 291, 146, 91, 181, 84,
         17, 314, 405, 321, 375, 78, 191, 80, 81, 82, 13, 312, 311, 310, 415,
         95, 88, 178, 87, 14, 317, 402, 318, 324, 308]
_HANDS = list(range(468, 489)) + list(range(522, 543))
_LANDMARKS = _LIPS + _HANDS
_N_COLS = 82
_TAIL0 = 536
_N_TAIL = _N_LMK - _TAIL0

_D_RUNS = []
_GROW_D = {}
_GD = 0
_DBANDS = [(468, 489), (522, _TAIL0)]
_DWINS = sorted({_l // 8 for _l in _LANDMARKS if _l < _TAIL0})
_i = 0
while _i < len(_DWINS):
    _j = _i
    while _j + 1 < len(_DWINS) and _DWINS[_j + 1] == _DWINS[_j] + 1:
        _j += 1
    _start, _nr = _DWINS[_i] * 8, (_DWINS[_j] - _DWINS[_i] + 1) * 8
    _is_hand = any(_start < hi and lo < _start + _nr for lo, hi in _DBANDS)
    _D_RUNS.append((_start, _nr, _GD, _is_hand))
    for _r in range(_start, _start + _nr):
        _GROW_D[_r] = _GD + _r - _start
    _GD += _nr
    _i = _j + 1
_G_TAIL = _GD
for _k in range(_N_TAIL):
    _GROW_D[_TAIL0 + _k] = _G_TAIL + _k
_GD += 8
_GTOT = 3 * _GD

_HAND_SEGS = []
for _d in range(3):
    for _lo, _hi in ((468, 489), (522, 543)):
        _r = _lo
        while _r < _hi:
            _g0, _ln = _GROW_D[_r], 1
            while _r + _ln < _hi and _GROW_D[_r + _ln] == _g0 + _ln:
                _ln += 1
            _HAND_SEGS.append((_d * _GD + _g0, _ln))
            _r += _ln
_SEL_SEGS = []
for _d in range(3):
    _gs = [_d * _GD + _GROW_D[_l] for _l in _LANDMARKS]
    _segs, _k = [], 0
    while _k < len(_gs):
        _g0, _ln = _gs[_k], 1
        while _k + _ln < len(_gs) and _gs[_k + _ln] == _g0 + _ln:
            _ln += 1
        _segs.append((_g0, _ln))
        _k += _ln
    _SEL_SEGS.append(_segs)


def _preprocess_kernel(x_ref, out_ref, nef_ref, g_ref, sem_h, sem_l):
    hand_copies = []
    for start, nr, g0, is_hand in _D_RUNS:
        if not is_hand:
            continue
        for d in range(3):
            c = pltpu.make_async_copy(
                x_ref.at[d, start:start + nr, :],
                g_ref.at[d * _GD + g0:d * _GD + g0 + nr, :], sem_h)
            c.start()
            hand_copies.append(c)
    for d in range(3):
        g0 = d * _GD + _G_TAIL
        g_ref[g0 + _N_TAIL:g0 + 8, :] = jnp.zeros((1, _N_FRAMES), jnp.float32)
        c = pltpu.make_async_copy(x_ref.at[d, _TAIL0:_N_LMK, :],
                                  g_ref.at[g0:g0 + _N_TAIL, :], sem_h)
        c.start()
        hand_copies.append(c)
    lip_copies = []
    for start, nr, g0, is_hand in _D_RUNS:
        if is_hand:
            continue
        for d in range(3):
            c = pltpu.make_async_copy(
                x_ref.at[d, start:start + nr, :],
                g_ref.at[d * _GD + g0:d * _GD + g0 + nr, :], sem_l)
            c.start()
            lip_copies.append(c)
    for c in hand_copies:
        c.wait()

    hs = jnp.zeros((1, _N_FRAMES), jnp.float32)
    for g0, ln in _HAND_SEGS:
        hs = hs + jnp.sum(g_ref[g0:g0 + ln, :], axis=0, keepdims=True)
    m = (hs > 0).astype(jnp.float32)

    p_incl = m
    k = 1
    while k < _N_FRAMES:
        shifted = jnp.concatenate(
            [jnp.zeros((1, k), jnp.float32), p_incl[:, :_N_FRAMES - k]], axis=1)
        p_incl = p_incl + shifted
        k *= 2
    p = p_incl - m
    n = lax.slice(p_incl, (0, _N_FRAMES - 1), (1, _N_FRAMES))

    pool = jnp.floor((2.0 * n + (_INPUT_SIZE - 1)) / _INPUT_SIZE)
    q = pool + 1.0
    pad_left = jnp.floor((pool * _INPUT_SIZE - 2.0 * n) / 2.0) + _INPUT_SIZE // 2
    total = _INPUT_SIZE * q

    i_col = lax.broadcasted_iota(
        jnp.int32, (_INPUT_SIZE, 1), 0).astype(jnp.float32)
    a = jnp.where(p == 0, 0.0, 2.0 * p + pad_left)
    b = jnp.where(p == n - 1.0, total, 2.0 * p + 2.0 + pad_left)
    lo = jnp.maximum(a, i_col * q)
    hi = jnp.minimum(b, (i_col + 1.0) * q)
    w_long = jnp.maximum(hi - lo, 0.0) / q
    w_short = (p == i_col).astype(jnp.float32)
    wt = m * jnp.where(n < _INPUT_SIZE, w_short, w_long)

    f_row = lax.broadcasted_iota(
        jnp.int32, (1, _N_FRAMES), 1).astype(jnp.float32)
    nef = jnp.sum(wt * f_row, axis=1, keepdims=True)
    nef_ref[...] = jnp.where((i_col < n) | (n >= _INPUT_SIZE), nef, -1.0)

    for c in lip_copies:
        c.wait()
    wfi = wt.T
    pooled = lax.dot_general(g_ref[...], wfi, (((1,), (0,)), ((), ())))
    pooled_t = pooled.T
    out_ref[...] = jnp.concatenate(
        [jnp.concatenate([pooled_t[:, g0:g0 + ln] for g0, ln in segs], axis=1)
         for segs in _SEL_SEGS], axis=0)


def kernel(data0):
    X = data0.transpose(2, 1, 0)
    out, nef = pl.pallas_call(
        _preprocess_kernel,
        in_specs=[pl.BlockSpec(memory_space=pl.ANY)],
        out_shape=(
            jax.ShapeDtypeStruct((3 * _INPUT_SIZE, _N_COLS), jnp.float32),
            jax.ShapeDtypeStruct((_INPUT_SIZE, 1), jnp.float32),
        ),
        scratch_shapes=[
            pltpu.VMEM((_GTOT, _N_FRAMES), jnp.float32),
            pltpu.SemaphoreType.DMA,
            pltpu.SemaphoreType.DMA,
        ],
    )(X)
    return (out.reshape(3, _INPUT_SIZE, _N_COLS).transpose(1, 2, 0),
            nef.reshape(_INPUT_SIZE))


# speedup vs baseline: 1.2294x; 1.1533x over previous
import jax
import jax.numpy as jnp
from jax import lax
from jax.experimental import pallas as pl
from jax.experimental.pallas import tpu as pltpu

_INPUT_SIZE = 64
_N_FRAMES = 2048
_N_LMK = 543
_LIPS = [61, 185, 40, 39, 37, 0, 267, 269, 270, 409, 291, 146, 91, 181, 84,
         17, 314, 405, 321, 375, 78, 191, 80, 81, 82, 13, 312, 311, 310, 415,
         95, 88, 178, 87, 14, 317, 402, 318, 324, 308]
_HANDS = list(range(468, 489)) + list(range(522, 543))
_LANDMARKS = _LIPS + _HANDS
_N_COLS = 82
_TAIL0 = 536
_N_TAIL = _N_LMK - _TAIL0

_D_RUNS = []
_GROW_D = {}
_GD = 0
_DBANDS = [(468, 489), (522, _TAIL0)]
_DWINS = sorted({_l // 8 for _l in _LANDMARKS if _l < _TAIL0})
_i = 0
while _i < len(_DWINS):
    _j = _i
    while _j + 1 < len(_DWINS) and _DWINS[_j + 1] == _DWINS[_j] + 1:
        _j += 1
    _start, _nr = _DWINS[_i] * 8, (_DWINS[_j] - _DWINS[_i] + 1) * 8
    _is_hand = any(_start < hi and lo < _start + _nr for lo, hi in _DBANDS)
    _D_RUNS.append((_start, _nr, _GD, _is_hand))
    for _r in range(_start, _start + _nr):
        _GROW_D[_r] = _GD + _r - _start
    _GD += _nr
    _i = _j + 1
_G_TAIL = _GD
for _k in range(_N_TAIL):
    _GROW_D[_TAIL0 + _k] = _G_TAIL + _k
_GD += 8
_GTOT = 3 * _GD

_HAND_SEGS = []
for _d in range(3):
    for _lo, _hi in ((468, 489), (522, 543)):
        _r = _lo
        while _r < _hi:
            _g0, _ln = _GROW_D[_r], 1
            while _r + _ln < _hi and _GROW_D[_r + _ln] == _g0 + _ln:
                _ln += 1
            _HAND_SEGS.append((_d * _GD + _g0, _ln))
            _r += _ln
_SEL_SEGS = []
for _d in range(3):
    _gs = [_d * _GD + _GROW_D[_l] for _l in _LANDMARKS]
    _segs, _k = [], 0
    while _k < len(_gs):
        _g0, _ln = _gs[_k], 1
        while _k + _ln < len(_gs) and _gs[_k + _ln] == _g0 + _ln:
            _ln += 1
        _segs.append((_g0, _ln))
        _k += _ln
    _SEL_SEGS.append(_segs)


def _preprocess_kernel(x_ref, out_ref, nef_ref, g_ref, sem_h, sem_l):
    hand_copies = []
    for start, nr, g0, is_hand in _D_RUNS:
        if not is_hand:
            continue
        for d in range(3):
            c = pltpu.make_async_copy(
                x_ref.at[d, start:start + nr, :],
                g_ref.at[d * _GD + g0:d * _GD + g0 + nr, :], sem_h)
            c.start()
            hand_copies.append(c)
    for d in range(3):
        g0 = d * _GD + _G_TAIL
        g_ref[g0 + _N_TAIL:g0 + 8, :] = jnp.zeros((1, _N_FRAMES), jnp.float32)
        c = pltpu.make_async_copy(x_ref.at[d, _TAIL0:_N_LMK, :],
                                  g_ref.at[g0:g0 + _N_TAIL, :], sem_h)
        c.start()
        hand_copies.append(c)
    lip_copies = []
    for start, nr, g0, is_hand in _D_RUNS:
        if is_hand:
            continue
        for d in range(3):
            c = pltpu.make_async_copy(
                x_ref.at[d, start:start + nr, :],
                g_ref.at[d * _GD + g0:d * _GD + g0 + nr, :], sem_l)
            c.start()
            lip_copies.append(c)
    for c in hand_copies:
        c.wait()

    hs = jnp.zeros((1, _N_FRAMES), jnp.float32)
    for g0, ln in _HAND_SEGS:
        hs = hs + jnp.sum(g_ref[g0:g0 + ln, :], axis=0, keepdims=True)
    m = (hs > 0).astype(jnp.float32)

    p_incl = m
    k = 1
    while k < _N_FRAMES:
        shifted = jnp.concatenate(
            [jnp.zeros((1, k), jnp.float32), p_incl[:, :_N_FRAMES - k]], axis=1)
        p_incl = p_incl + shifted
        k *= 2
    p = p_incl - m
    n = lax.slice(p_incl, (0, _N_FRAMES - 1), (1, _N_FRAMES))

    pool = jnp.floor((2.0 * n + (_INPUT_SIZE - 1)) / _INPUT_SIZE)
    q = pool + 1.0
    pad_left = jnp.floor((pool * _INPUT_SIZE - 2.0 * n) / 2.0) + _INPUT_SIZE // 2
    total = _INPUT_SIZE * q

    i_col = lax.broadcasted_iota(
        jnp.int32, (_INPUT_SIZE, 1), 0).astype(jnp.float32)
    a = jnp.where(p == 0, 0.0, 2.0 * p + pad_left)
    b = jnp.where(p == n - 1.0, total, 2.0 * p + 2.0 + pad_left)
    lo = jnp.maximum(a, i_col * q)
    hi = jnp.minimum(b, (i_col + 1.0) * q)
    w_long = jnp.maximum(hi - lo, 0.0) / q
    w_short = (p == i_col).astype(jnp.float32)
    wt = m * jnp.where(n < _INPUT_SIZE, w_short, w_long)

    f_row = lax.broadcasted_iota(
        jnp.int32, (1, _N_FRAMES), 1).astype(jnp.float32)
    nef = jnp.sum(wt * f_row, axis=1, keepdims=True)
    nef_ref[...] = jnp.where((i_col < n) | (n >= _INPUT_SIZE), nef, -1.0)

    for c in lip_copies:
        c.wait()
    wfi = wt.T
    pooled = lax.dot_general(g_ref[...], wfi, (((1,), (0,)), ((), ())))
    out_ref[...] = jnp.concatenate(
        [jnp.concatenate([pooled[g0:g0 + ln, :] for g0, ln in segs],
                         axis=0).T
         for segs in _SEL_SEGS], axis=0)


def kernel(data0):
    X = data0.transpose(2, 1, 0)
    out, nef = pl.pallas_call(
        _preprocess_kernel,
        in_specs=[pl.BlockSpec(memory_space=pl.ANY)],
        out_shape=(
            jax.ShapeDtypeStruct((3 * _INPUT_SIZE, _N_COLS), jnp.float32),
            jax.ShapeDtypeStruct((_INPUT_SIZE, 1), jnp.float32),
        ),
        scratch_shapes=[
            pltpu.VMEM((_GTOT, _N_FRAMES), jnp.float32),
            pltpu.SemaphoreType.DMA,
            pltpu.SemaphoreType.DMA,
        ],
    )(X)
    return (out.reshape(3, _INPUT_SIZE, _N_COLS).transpose(1, 2, 0),
            nef.reshape(_INPUT_SIZE))
